# baseline (device time: 185367 ns/iter reference)
import jax
import jax.numpy as jnp
from jax import lax
from jax.experimental import pallas as pl
from jax.experimental.pallas import tpu as pltpu

N_DEV = 4
E_LOCAL = 8
E_TOTAL = 32
N_TOK = 2048
D = 512
H = 1024
CHUNK = N_TOK // N_DEV


def kernel(x, router_W, route_idx, expert_W, shared_W):
    def body(x_ref, rW_ref, idx_ref, eW_ref, sW_ref, out_ref,
             recv_buf, send_sems, recv_sems):
        p = lax.axis_index("i")
        left = lax.rem(p - 1 + N_DEV, N_DEV)
        right = lax.rem(p + 1, N_DEV)

        xv = x_ref[:, :]
        scores = jnp.dot(xv, rW_ref[:, :],
                         preferred_element_type=jnp.float32)
        m = jnp.max(scores, axis=1, keepdims=True)
        ex = jnp.exp(scores - m)
        probs = ex / jnp.sum(ex, axis=1, keepdims=True)
        ridx = idx_ref[:, :]
        iota = lax.broadcasted_iota(jnp.int32, (N_TOK, E_TOTAL), 1)
        p_sel = jnp.sum(jnp.where(ridx == iota, probs, 0.0),
                        axis=1, keepdims=True)

        for j in range(E_LOCAL):
            e = p * E_LOCAL + j
            w = jnp.where(ridx == e, p_sel, 0.0)
            xs = xv * w
            contrib = jnp.dot(xs, eW_ref[j],
                              preferred_element_type=jnp.float32)
            if j == 0:
                out_ref[:, :] = contrib
            else:
                out_ref[:, :] += contrib

        xq = x_ref[pl.ds(p * CHUNK, CHUNK), :]
        sq = jnp.dot(xq, sW_ref[:, :], preferred_element_type=jnp.float32)
        out_ref[pl.ds(p * CHUNK, CHUNK), :] += sq

        barrier = pltpu.get_barrier_semaphore()
        for nbr in [left, right]:
            pl.semaphore_signal(barrier, inc=1, device_id=(nbr,),
                                device_id_type=pl.DeviceIdType.MESH)
        pl.semaphore_wait(barrier, 2)

        for s in range(N_DEV - 1):
            sc = lax.rem(p - s + N_DEV, N_DEV)
            rdma = pltpu.make_async_remote_copy(
                src_ref=out_ref.at[pl.ds(sc * CHUNK, CHUNK)],
                dst_ref=recv_buf.at[s],
                send_sem=send_sems.at[s],
                recv_sem=recv_sems.at[s],
                device_id=(right,),
                device_id_type=pl.DeviceIdType.MESH,
            )
            rdma.start()
            rdma.wait()
            ac = lax.rem(p - 1 - s + N_DEV, N_DEV)
            out_ref[pl.ds(ac * CHUNK, CHUNK), :] += recv_buf[s]

        for s in range(N_DEV - 1):
            gc = lax.rem(p + 1 - s + N_DEV, N_DEV)
            rdma = pltpu.make_async_remote_copy(
                src_ref=out_ref.at[pl.ds(gc * CHUNK, CHUNK)],
                dst_ref=out_ref.at[pl.ds(gc * CHUNK, CHUNK)],
                send_sem=send_sems.at[N_DEV - 1 + s],
                recv_sem=recv_sems.at[N_DEV - 1 + s],
                device_id=(right,),
                device_id_type=pl.DeviceIdType.MESH,
            )
            rdma.start()
            rdma.wait()

    return pl.pallas_call(
        body,
        out_shape=jax.ShapeDtypeStruct((N_TOK, H), jnp.float32),
        in_specs=[pl.BlockSpec(memory_space=pltpu.VMEM)] * 5,
        out_specs=pl.BlockSpec(memory_space=pltpu.VMEM),
        scratch_shapes=[
            pltpu.VMEM((N_DEV - 1, CHUNK, H), jnp.float32),
            pltpu.SemaphoreType.DMA((2 * (N_DEV - 1),)),
            pltpu.SemaphoreType.DMA((2 * (N_DEV - 1),)),
        ],
        compiler_params=pltpu.CompilerParams(collective_id=0),
    )(x, router_W, route_idx, expert_W, shared_W)


# device time: 102127 ns/iter; 1.8151x vs baseline; 1.8151x over previous
import jax
import jax.numpy as jnp
from jax import lax
from jax.experimental import pallas as pl
from jax.experimental.pallas import tpu as pltpu

N_DEV = 4
E_LOCAL = 8
E_TOTAL = 32
N_TOK = 2048
D = 512
H = 1024
CHUNK = N_TOK // N_DEV
HALF = CHUNK // 2


def kernel(x, router_W, route_idx, expert_W, shared_W):
    def body(x_ref, rW_ref, idx_ref, eW_ref, sW_ref, out_ref,
             recv_r, recv_l, send_sems, recv_sems):
        p = lax.axis_index("i")
        left = lax.rem(p - 1 + N_DEV, N_DEV)
        right = lax.rem(p + 1, N_DEV)

        def compute_chunk(c, add_shared=False):
            xq = x_ref[pl.ds(c * CHUNK, CHUNK), :]
            idxq = idx_ref[pl.ds(c * CHUNK, CHUNK), :]
            scores = jnp.dot(xq, rW_ref[:, :],
                             preferred_element_type=jnp.float32)
            mx = jnp.max(scores, axis=1, keepdims=True)
            exs = jnp.exp(scores - mx)
            probs = exs / jnp.sum(exs, axis=1, keepdims=True)
            iota = lax.broadcasted_iota(jnp.int32, (CHUNK, E_TOTAL), 1)
            psq = jnp.sum(jnp.where(idxq == iota, probs, 0.0),
                          axis=1, keepdims=True)
            acc = None
            for j in range(E_LOCAL):
                e = p * E_LOCAL + j
                w = jnp.where(idxq == e, psq, 0.0)
                contrib = jnp.dot(xq * w, eW_ref[j],
                                  preferred_element_type=jnp.float32)
                acc = contrib if acc is None else acc + contrib
            if add_shared:
                acc = acc + jnp.dot(xq, sW_ref[:, :],
                                    preferred_element_type=jnp.float32)
            out_ref[pl.ds(c * CHUNK, CHUNK), :] = acc

        def rs_rdma(s):
            sc_r = lax.rem(p - s + N_DEV, N_DEV)
            sc_l = lax.rem(p + s, N_DEV)
            r = pltpu.make_async_remote_copy(
                src_ref=out_ref.at[pl.ds(sc_r * CHUNK, HALF)],
                dst_ref=recv_r.at[s],
                send_sem=send_sems.at[s],
                recv_sem=recv_sems.at[s],
                device_id=(right,),
                device_id_type=pl.DeviceIdType.MESH,
            )
            l = pltpu.make_async_remote_copy(
                src_ref=out_ref.at[pl.ds(sc_l * CHUNK + HALF, HALF)],
                dst_ref=recv_l.at[s],
                send_sem=send_sems.at[3 + s],
                recv_sem=recv_sems.at[3 + s],
                device_id=(left,),
                device_id_type=pl.DeviceIdType.MESH,
            )
            return r, l

        def rs_add(s):
            ac_r = lax.rem(p - 1 - s + N_DEV, N_DEV)
            ac_l = lax.rem(p + 1 + s, N_DEV)
            out_ref[pl.ds(ac_r * CHUNK, HALF), :] += recv_r[s]
            out_ref[pl.ds(ac_l * CHUNK + HALF, HALF), :] += recv_l[s]

        cp = p
        compute_chunk(cp, add_shared=True)

        barrier = pltpu.get_barrier_semaphore()
        for nbr in [left, right]:
            pl.semaphore_signal(barrier, inc=1, device_id=(nbr,),
                                device_id_type=pl.DeviceIdType.MESH)
        pl.semaphore_wait(barrier, 2)

        r0, l0 = rs_rdma(0)
        r0.start()
        l0.start()
        compute_chunk(lax.rem(p + 1, N_DEV))
        compute_chunk(lax.rem(p - 1 + N_DEV, N_DEV))
        r0.wait()
        l0.wait()
        rs_add(0)

        r1, l1 = rs_rdma(1)
        r1.start()
        l1.start()
        compute_chunk(lax.rem(p + 2, N_DEV))
        r1.wait()
        l1.wait()
        rs_add(1)

        r2, l2 = rs_rdma(2)
        r2.start()
        l2.start()
        r2.wait()
        l2.wait()
        rs_add(2)

        for s in range(N_DEV - 1):
            gc_r = lax.rem(p + 1 - s + N_DEV, N_DEV)
            gc_l = lax.rem(p - 1 + s + N_DEV, N_DEV)
            r = pltpu.make_async_remote_copy(
                src_ref=out_ref.at[pl.ds(gc_r * CHUNK, HALF)],
                dst_ref=out_ref.at[pl.ds(gc_r * CHUNK, HALF)],
                send_sem=send_sems.at[6 + s],
                recv_sem=recv_sems.at[6 + s],
                device_id=(right,),
                device_id_type=pl.DeviceIdType.MESH,
            )
            l = pltpu.make_async_remote_copy(
                src_ref=out_ref.at[pl.ds(gc_l * CHUNK + HALF, HALF)],
                dst_ref=out_ref.at[pl.ds(gc_l * CHUNK + HALF, HALF)],
                send_sem=send_sems.at[9 + s],
                recv_sem=recv_sems.at[9 + s],
                device_id=(left,),
                device_id_type=pl.DeviceIdType.MESH,
            )
            r.start()
            l.start()
            r.wait()
            l.wait()

    return pl.pallas_call(
        body,
        out_shape=jax.ShapeDtypeStruct((N_TOK, H), jnp.float32),
        in_specs=[pl.BlockSpec(memory_space=pltpu.VMEM)] * 5,
        out_specs=pl.BlockSpec(memory_space=pltpu.VMEM),
        scratch_shapes=[
            pltpu.VMEM((N_DEV - 1, HALF, H), jnp.float32),
            pltpu.VMEM((N_DEV - 1, HALF, H), jnp.float32),
            pltpu.SemaphoreType.DMA((12,)),
            pltpu.SemaphoreType.DMA((12,)),
        ],
        compiler_params=pltpu.CompilerParams(collective_id=0),
    )(x, router_W, route_idx, expert_W, shared_W)


# device time: 72626 ns/iter; 2.5524x vs baseline; 1.4062x over previous
import jax
import jax.numpy as jnp
from jax import lax
from jax.experimental import pallas as pl
from jax.experimental.pallas import tpu as pltpu

N_DEV = 4
E_LOCAL = 8
E_TOTAL = 32
N_TOK = 2048
D = 512
H = 1024
CHUNK = N_TOK // N_DEV
HALF = CHUNK // 2


def kernel(x, router_W, route_idx, expert_W, shared_W):
    def body(x_ref, rW_ref, idx_ref, eW_ref, sW_ref, out_ref,
             pbuf, recv_r, recv_l, send_sems, recv_sems):
        p = lax.axis_index("i")
        left = lax.rem(p - 1 + N_DEV, N_DEV)
        right = lax.rem(p + 1, N_DEV)

        def compute_chunk(c, add_shared=False):
            xq = x_ref[pl.ds(c * CHUNK, CHUNK), :]
            idxq = idx_ref[pl.ds(c * CHUNK, CHUNK), :]
            scores = jnp.dot(xq, rW_ref[:, :],
                             preferred_element_type=jnp.float32)
            mx = jnp.max(scores, axis=1, keepdims=True)
            exs = jnp.exp(scores - mx)
            probs = exs / jnp.sum(exs, axis=1, keepdims=True)
            iota = lax.broadcasted_iota(jnp.int32, (CHUNK, E_TOTAL), 1)
            psq = jnp.sum(jnp.where(idxq == iota, probs, 0.0),
                          axis=1, keepdims=True)
            acc = None
            for j in range(E_LOCAL):
                e = p * E_LOCAL + j
                w = jnp.where(idxq == e, psq, 0.0)
                contrib = jnp.dot(xq * w, eW_ref[j],
                                  preferred_element_type=jnp.float32)
                acc = contrib if acc is None else acc + contrib
            if add_shared:
                acc = acc + jnp.dot(xq, sW_ref[:, :],
                                    preferred_element_type=jnp.float32)
            pbuf[pl.ds(c * CHUNK, CHUNK), :] = acc.astype(jnp.bfloat16)

        def rs_rdma(s):
            sc_r = lax.rem(p - s + N_DEV, N_DEV)
            sc_l = lax.rem(p + s, N_DEV)
            r = pltpu.make_async_remote_copy(
                src_ref=pbuf.at[pl.ds(sc_r * CHUNK, HALF)],
                dst_ref=recv_r.at[s],
                send_sem=send_sems.at[s],
                recv_sem=recv_sems.at[s],
                device_id=(right,),
                device_id_type=pl.DeviceIdType.MESH,
            )
            l = pltpu.make_async_remote_copy(
                src_ref=pbuf.at[pl.ds(sc_l * CHUNK + HALF, HALF)],
                dst_ref=recv_l.at[s],
                send_sem=send_sems.at[3 + s],
                recv_sem=recv_sems.at[3 + s],
                device_id=(left,),
                device_id_type=pl.DeviceIdType.MESH,
            )
            return r, l

        def rs_add(s):
            ac_r = lax.rem(p - 1 - s + N_DEV, N_DEV)
            ac_l = lax.rem(p + 1 + s, N_DEV)
            pbuf[pl.ds(ac_r * CHUNK, HALF), :] += recv_r[s]
            pbuf[pl.ds(ac_l * CHUNK + HALF, HALF), :] += recv_l[s]

        def emit(c, half):
            rows = c * CHUNK + half * HALF
            out_ref[pl.ds(rows, HALF), :] = pbuf[pl.ds(rows, HALF), :].astype(
                jnp.float32)

        compute_chunk(p, add_shared=True)

        barrier = pltpu.get_barrier_semaphore()
        for nbr in [left, right]:
            pl.semaphore_signal(barrier, inc=1, device_id=(nbr,),
                                device_id_type=pl.DeviceIdType.MESH)
        pl.semaphore_wait(barrier, 2)

        r0, l0 = rs_rdma(0)
        r0.start()
        l0.start()
        compute_chunk(lax.rem(p + 1, N_DEV))
        compute_chunk(lax.rem(p - 1 + N_DEV, N_DEV))
        r0.wait()
        l0.wait()
        rs_add(0)

        r1, l1 = rs_rdma(1)
        r1.start()
        l1.start()
        compute_chunk(lax.rem(p + 2, N_DEV))
        r1.wait()
        l1.wait()
        rs_add(1)

        r2, l2 = rs_rdma(2)
        r2.start()
        l2.start()
        r2.wait()
        l2.wait()
        rs_add(2)

        ag = []
        for s in range(N_DEV - 1):
            gc_r = lax.rem(p + 1 - s + N_DEV, N_DEV)
            gc_l = lax.rem(p - 1 + s + N_DEV, N_DEV)
            r = pltpu.make_async_remote_copy(
                src_ref=pbuf.at[pl.ds(gc_r * CHUNK, HALF)],
                dst_ref=pbuf.at[pl.ds(gc_r * CHUNK, HALF)],
                send_sem=send_sems.at[6 + s],
                recv_sem=recv_sems.at[6 + s],
                device_id=(right,),
                device_id_type=pl.DeviceIdType.MESH,
            )
            l = pltpu.make_async_remote_copy(
                src_ref=pbuf.at[pl.ds(gc_l * CHUNK + HALF, HALF)],
                dst_ref=pbuf.at[pl.ds(gc_l * CHUNK + HALF, HALF)],
                send_sem=send_sems.at[9 + s],
                recv_sem=recv_sems.at[9 + s],
                device_id=(left,),
                device_id_type=pl.DeviceIdType.MESH,
            )
            r.start()
            l.start()
            if s == 0:
                emit(lax.rem(p + 1, N_DEV), 0)
                emit(lax.rem(p - 1 + N_DEV, N_DEV), 1)
            elif s == 1:
                emit(p, 0)
                emit(p, 1)
            else:
                emit(lax.rem(p - 1 + N_DEV, N_DEV), 0)
                emit(lax.rem(p + 1, N_DEV), 1)
            r.wait()
            l.wait()
        emit(lax.rem(p + 2, N_DEV), 0)
        emit(lax.rem(p + 2, N_DEV), 1)

    return pl.pallas_call(
        body,
        out_shape=jax.ShapeDtypeStruct((N_TOK, H), jnp.float32),
        in_specs=[pl.BlockSpec(memory_space=pltpu.VMEM)] * 5,
        out_specs=pl.BlockSpec(memory_space=pltpu.VMEM),
        scratch_shapes=[
            pltpu.VMEM((N_TOK, H), jnp.bfloat16),
            pltpu.VMEM((N_DEV - 1, HALF, H), jnp.bfloat16),
            pltpu.VMEM((N_DEV - 1, HALF, H), jnp.bfloat16),
            pltpu.SemaphoreType.DMA((12,)),
            pltpu.SemaphoreType.DMA((12,)),
        ],
        compiler_params=pltpu.CompilerParams(collective_id=0),
    )(x, router_W, route_idx, expert_W, shared_W)
